# Initial kernel scaffold; baseline (speedup 1.0000x reference)
import jax
import jax.numpy as jnp
from jax.experimental import pallas as pl


def kernel(node_attrs, node_feats, edge_attrs, edge_feats, idx_i, idx_j, W1_0, W1_1, A0, A1, A2, A3, W2_0, W2_1, Wsk0, Wsk1):
    raise NotImplementedError("write your pallas kernel here")


# speedup vs baseline: 5.7367x; 5.7367x over previous
import functools

import jax
import jax.numpy as jnp
import numpy as np
from jax import lax
from jax.experimental import pallas as pl
from jax.experimental.pallas import tpu as pltpu
from jax.experimental.pallas import tpu_sc as plsc

N_NODES = 10000
N_EDGES = 160000
C = 128
OUT = 128
NSP = 10
NB = 8
HID = 64
NPATHS = 5
AVG = 16.0
G = 8
CG = C // G
SEC = 11
ROW = SEC * CG

NB_BLK = 1000
EB_BLK = 2000

SILU_SCALE = 1.6765324703310907


def _silu(x):
    return SILU_SCALE * x * (1.0 / (1.0 + jnp.exp(-x)))


def _node_kernel(nf_ref, attrs_ref, w10_ref, w11_ref, wsk0_ref, wsk1_ref,
                 sc_ref, ytab_ref):
    x0 = nf_ref[0]
    y0 = jnp.dot(x0, w10_ref[...], preferred_element_type=jnp.float32)
    ys = [y0]
    for d in range(3):
        ys.append(jnp.dot(nf_ref[1 + d], w11_ref[...],
                          preferred_element_type=jnp.float32))
    for g in range(G):
        for comp in range(4):
            ytab_ref[g, :, comp * CG:(comp + 1) * CG] = \
                ys[comp][:, g * CG:(g + 1) * CG]
    for comp in range(4):
        x = nf_ref[comp]
        wsk = wsk0_ref if comp == 0 else wsk1_ref
        acc = jnp.zeros((NB_BLK, C), jnp.float32)
        for v in range(NSP):
            acc += jnp.dot(x * attrs_ref[:, v:v + 1], wsk[v],
                           preferred_element_type=jnp.float32)
        sc_ref[comp] = acc


def _run_node_kernel(nf_t, attrs, w10, w11, wsk0_t, wsk1_t):
    grid = (N_NODES // NB_BLK,)
    return pl.pallas_call(
        _node_kernel,
        grid=grid,
        in_specs=[
            pl.BlockSpec((4, NB_BLK, C), lambda i: (0, i, 0)),
            pl.BlockSpec((NB_BLK, NSP), lambda i: (i, 0)),
            pl.BlockSpec((C, C), lambda i: (0, 0)),
            pl.BlockSpec((C, C), lambda i: (0, 0)),
            pl.BlockSpec((NSP, C, C), lambda i: (0, 0, 0)),
            pl.BlockSpec((NSP, C, C), lambda i: (0, 0, 0)),
        ],
        out_specs=[
            pl.BlockSpec((4, NB_BLK, C), lambda i: (0, i, 0)),
            pl.BlockSpec((G, NB_BLK, 4 * CG), lambda i: (0, i, 0)),
        ],
        out_shape=[
            jax.ShapeDtypeStruct((4, N_NODES, C), jnp.float32),
            jax.ShapeDtypeStruct((G, N_NODES, 4 * CG), jnp.float32),
        ],
    )(nf_t, attrs, w10, w11, wsk0_t, wsk1_t)


def _edge_kernel(ef_ref, a0_ref, a1_ref, a2_ref, a3_ref, tpw_ref):
    h = _silu(jnp.dot(ef_ref[...], a0_ref[...],
                      preferred_element_type=jnp.float32))
    h = _silu(jnp.dot(h, a1_ref[...], preferred_element_type=jnp.float32))
    h = _silu(jnp.dot(h, a2_ref[...], preferred_element_type=jnp.float32))
    t = jnp.dot(h, a3_ref[...], preferred_element_type=jnp.float32)
    for g in range(G):
        tpw_ref[g] = t[:, g * NPATHS * CG:(g + 1) * NPATHS * CG]


def _run_edge_kernel(ef, a0t, a1t, a2t, a3pt):
    grid = (N_EDGES // EB_BLK,)
    return pl.pallas_call(
        _edge_kernel,
        grid=grid,
        in_specs=[
            pl.BlockSpec((EB_BLK, NB), lambda i: (i, 0)),
            pl.BlockSpec((NB, HID), lambda i: (0, 0)),
            pl.BlockSpec((HID, HID), lambda i: (0, 0)),
            pl.BlockSpec((HID, HID), lambda i: (0, 0)),
            pl.BlockSpec((HID, NPATHS * C), lambda i: (0, 0)),
        ],
        out_specs=[
            pl.BlockSpec((G, EB_BLK, NPATHS * CG), lambda i: (0, i, 0)),
        ],
        out_shape=[
            jax.ShapeDtypeStruct((G, N_EDGES, NPATHS * CG), jnp.float32),
        ],
    )(ef, a0t, a1t, a2t, a3pt)[0]


B_EDGE = 40
B_IDX = 48
EPT = N_EDGES // 16
NCHUNK = EPT // B_EDGE
N_PAD = 10240
RPT = N_PAD // 16
ZR = 16


def _sc_body(ytab_h, tpw_h, rec_h, idxj_h, idxi_h, out_h,
             acc, rec_v, tpw_v, idxj_v, idxi_v, idx2_v, z_v, p_v, zero_v,
             sem_in, sem_g, sem_j):
    cid = lax.axis_index("c")
    sid = lax.axis_index("s")

    def zinit(t, _):
        r = t // SEC
        k = t % SEC
        zero_v[r, pl.ds(k * CG, CG)] = jnp.zeros((CG,), jnp.float32)
        return 0
    lax.fori_loop(0, ZR * SEC, zinit, 0)
    idxj_v[pl.ds(2 * 16, 16)] = jnp.zeros((16,), jnp.int32)

    for gi in range(G // 2):
        g = cid * (G // 2) + gi

        for r5 in range(RPT // ZR):
            pltpu.sync_copy(zero_v, acc.at[pl.ds(sid * RPT + r5 * ZR, ZR)])
        plsc.subcore_barrier()

        def chunk(ch, _):
            base = sid * EPT + ch * B_EDGE
            c1 = pltpu.async_copy(rec_h.at[pl.ds(base, B_EDGE)], rec_v, sem_in)
            c2 = pltpu.async_copy(tpw_h.at[g, pl.ds(base, B_EDGE)], tpw_v,
                                  sem_in)
            c3 = pltpu.async_copy(idxj_h.at[pl.ds(base, B_EDGE)],
                                  idxj_v.at[pl.ds(0, B_EDGE)], sem_j)
            c4 = pltpu.async_copy(idxi_h.at[pl.ds(base, B_EDGE)], idxi_v,
                                  sem_in)
            c3.wait()

            def addb(k, _):
                idx2_v[pl.ds(k * 16, 16)] = \
                    idxj_v[pl.ds(k * 16, 16)] + g * N_NODES
                return 0
            lax.fori_loop(0, B_IDX // 16, addb, 0)
            cg_ = pltpu.async_copy(ytab_h.at[idx2_v], z_v, sem_g)
            c1.wait()
            c2.wait()
            c4.wait()
            cg_.wait()

            def edge(e, _):
                recrow = rec_v[e, pl.ds(0, 16)]
                a0 = recrow[0]
                a10 = recrow[1]
                a11 = recrow[2]
                a12 = recrow[3]
                z0 = z_v[e, pl.ds(0, CG)]
                z10 = z_v[e, pl.ds(CG, CG)]
                z11 = z_v[e, pl.ds(2 * CG, CG)]
                z12 = z_v[e, pl.ds(3 * CG, CG)]
                w0 = tpw_v[e, pl.ds(0, CG)]
                w1 = tpw_v[e, pl.ds(CG, CG)]
                w2 = tpw_v[e, pl.ds(2 * CG, CG)]
                w3 = tpw_v[e, pl.ds(3 * CG, CG)]
                w4 = tpw_v[e, pl.ds(4 * CG, CG)]
                p_v[e, pl.ds(0, CG)] = w0 * z0 * a0
                p_v[e, pl.ds(CG, CG)] = w3 * (z10 * a10 + z11 * a11
                                              + z12 * a12)
                u = w1 * z0
                p_v[e, pl.ds(2 * CG, CG)] = u * a10
                p_v[e, pl.ds(3 * CG, CG)] = u * a11
                p_v[e, pl.ds(4 * CG, CG)] = u * a12
                w2s = w2 * a0
                p_v[e, pl.ds(5 * CG, CG)] = w2s * z10
                p_v[e, pl.ds(6 * CG, CG)] = w2s * z11
                p_v[e, pl.ds(7 * CG, CG)] = w2s * z12
                t0 = (z10 * recrow[4] + z11 * recrow[7]
                      + z12 * recrow[10])
                t1 = (z10 * recrow[5] + z11 * recrow[8]
                      + z12 * recrow[11])
                t2 = (z10 * recrow[6] + z11 * recrow[9]
                      + z12 * recrow[12])
                p_v[e, pl.ds(8 * CG, CG)] = w4 * t0
                p_v[e, pl.ds(9 * CG, CG)] = w4 * t1
                p_v[e, pl.ds(10 * CG, CG)] = w4 * t2
                return 0
            lax.fori_loop(0, B_EDGE, edge, 0)
            pltpu.sync_copy(p_v, acc.at[idxi_v], add=True)
            return 0
        lax.fori_loop(0, NCHUNK, chunk, 0)
        plsc.subcore_barrier()
        pltpu.sync_copy(acc.at[pl.ds(sid * RPT, RPT)],
                        out_h.at[g, pl.ds(sid * RPT, RPT)])
        if gi < G // 2 - 1:
            plsc.subcore_barrier()


def _run_sc_stage(ytab_flat, tpw, rec, idxj, idxi):
    mesh = plsc.VectorSubcoreMesh(core_axis_name="c", subcore_axis_name="s")
    fn = functools.partial(
        pl.kernel,
        out_type=jax.ShapeDtypeStruct((G, N_PAD, ROW), jnp.float32),
        mesh=mesh,
        compiler_params=pltpu.CompilerParams(use_tc_tiling_on_sc=False),
        scratch_types=[
            pltpu.VMEM_SHARED((N_PAD, ROW), jnp.float32),
            pltpu.VMEM((B_EDGE, 16), jnp.float32),
            pltpu.VMEM((B_EDGE, NPATHS * CG), jnp.float32),
            pltpu.VMEM((B_IDX,), jnp.int32),
            pltpu.VMEM((B_EDGE,), jnp.int32),
            pltpu.VMEM((B_IDX,), jnp.int32),
            pltpu.VMEM((B_IDX, 4 * CG), jnp.float32),
            pltpu.VMEM((B_EDGE, ROW), jnp.float32),
            pltpu.VMEM((ZR, ROW), jnp.float32),
            pltpu.SemaphoreType.DMA,
            pltpu.SemaphoreType.DMA,
            pltpu.SemaphoreType.DMA,
        ],
    )(_sc_body)
    return fn(ytab_flat, tpw, rec, idxj, idxi)


def _out_kernel(msg_ref, wbig_ref, o_ref):
    acc = jnp.zeros((NB_BLK, OUT * 4), jnp.float32)
    for g in range(G):
        acc += jnp.dot(msg_ref[g], wbig_ref[g],
                       preferred_element_type=jnp.float32)
    o_ref[...] = acc


def _run_out_kernel(msg, wbig):
    grid = (N_NODES // NB_BLK,)
    return pl.pallas_call(
        _out_kernel,
        grid=grid,
        in_specs=[
            pl.BlockSpec((G, NB_BLK, ROW), lambda i: (0, i, 0)),
            pl.BlockSpec((G, ROW, OUT * 4), lambda i: (0, 0, 0)),
        ],
        out_specs=[pl.BlockSpec((NB_BLK, OUT * 4), lambda i: (i, 0))],
        out_shape=[jax.ShapeDtypeStruct((N_NODES, OUT * 4), jnp.float32)],
    )(msg, wbig)[0]


def _build_wbig(W2_0, W2_1):
    s0 = 1.0 / (np.sqrt(2 * C) * AVG)
    s1 = 1.0 / (np.sqrt(3 * C) * AVG)
    wb = jnp.zeros((G, ROW, OUT * 4), jnp.float32)
    part = W2_0[:, :C].reshape(OUT, G, CG).transpose(1, 2, 0) * s0
    wb = wb.at[:, 0:CG, 0::4].set(part)
    part = W2_0[:, C:].reshape(OUT, G, CG).transpose(1, 2, 0) * s0
    wb = wb.at[:, CG:2 * CG, 0::4].set(part)
    for pi in range(3):
        part = (W2_1[:, pi * C:(pi + 1) * C]
                .reshape(OUT, G, CG).transpose(1, 2, 0) * s1)
        for d in range(3):
            sec = 2 + pi * 3 + d
            wb = wb.at[:, sec * CG:(sec + 1) * CG, (1 + d)::4].set(part)
    return wb


def _prep(node_feats, edge_attrs, idx_i, idx_j,
          W1_0, W1_1, A0, A1, A2, A3, W2_0, W2_1, Wsk0, Wsk1):
    nf_t = jnp.transpose(node_feats, (2, 0, 1))
    rec = jnp.pad(edge_attrs, ((0, 0), (0, 3)))
    idxj = idx_j.astype(jnp.int32)
    idxi = idx_i.astype(jnp.int32)
    w10 = W1_0.T / np.sqrt(C)
    w11 = W1_1.T / np.sqrt(C)
    norm_sk = 1.0 / np.sqrt(C * NSP)
    wsk0_t = jnp.transpose(Wsk0, (2, 1, 0)) * norm_sk
    wsk1_t = jnp.transpose(Wsk1, (2, 1, 0)) * norm_sk
    a0t = A0.T / np.sqrt(NB)
    a1t = A1.T / np.sqrt(HID)
    a2t = A2.T / np.sqrt(HID)
    a3p = (A3.reshape(NPATHS, G, CG, HID)
           .transpose(1, 0, 2, 3).reshape(NPATHS * C, HID))
    a3pt = a3p.T / np.sqrt(HID)
    wbig = _build_wbig(W2_0, W2_1)
    return (nf_t, rec, idxj, idxi, w10, w11, wsk0_t, wsk1_t,
            a0t, a1t, a2t, a3pt, wbig)


def kernel(node_attrs, node_feats, edge_attrs, edge_feats, idx_i, idx_j,
           W1_0, W1_1, A0, A1, A2, A3, W2_0, W2_1, Wsk0, Wsk1):
    (nf_t, rec, idxj, idxi, w10, w11, wsk0_t, wsk1_t,
     a0t, a1t, a2t, a3pt, wbig) = _prep(
        node_feats, edge_attrs, idx_i, idx_j,
        W1_0, W1_1, A0, A1, A2, A3, W2_0, W2_1, Wsk0, Wsk1)

    sc_t, ytab = _run_node_kernel(nf_t, node_attrs, w10, w11, wsk0_t, wsk1_t)
    tpw = _run_edge_kernel(edge_feats, a0t, a1t, a2t, a3pt)
    ytab_flat = ytab.reshape(G * N_NODES, 4 * CG)
    msg = _run_sc_stage(ytab_flat, tpw, rec, idxj, idxi)
    oflat = _run_out_kernel(msg, wbig)

    message = oflat.reshape(N_NODES, OUT, 4)
    sc = jnp.transpose(sc_t, (1, 2, 0))
    return (message, sc)


# speedup vs baseline: 5.7394x; 1.0005x over previous
import functools

import jax
import jax.numpy as jnp
import numpy as np
from jax import lax
from jax.experimental import pallas as pl
from jax.experimental.pallas import tpu as pltpu
from jax.experimental.pallas import tpu_sc as plsc

N_NODES = 10000
N_EDGES = 160000
C = 128
OUT = 128
NSP = 10
NB = 8
HID = 64
NPATHS = 5
AVG = 16.0
G = 8
CG = C // G
SEC = 11
ROW = SEC * CG

NB_BLK = 1000
EB_BLK = 2000

SILU_SCALE = 1.6765324703310907


def _silu(x):
    return SILU_SCALE * x * (1.0 / (1.0 + jnp.exp(-x)))


def _node_kernel(nf_ref, attrs_ref, w10_ref, w11_ref, wsk0_ref, wsk1_ref,
                 sc_ref, ytab_ref):
    x0 = nf_ref[0]
    y0 = jnp.dot(x0, w10_ref[...], preferred_element_type=jnp.float32)
    ys = [y0]
    for d in range(3):
        ys.append(jnp.dot(nf_ref[1 + d], w11_ref[...],
                          preferred_element_type=jnp.float32))
    for g in range(G):
        for comp in range(4):
            ytab_ref[g, :, comp * CG:(comp + 1) * CG] = \
                ys[comp][:, g * CG:(g + 1) * CG]
    for comp in range(4):
        x = nf_ref[comp]
        wsk = wsk0_ref if comp == 0 else wsk1_ref
        acc = jnp.zeros((NB_BLK, C), jnp.float32)
        for v in range(NSP):
            acc += jnp.dot(x * attrs_ref[:, v:v + 1], wsk[v],
                           preferred_element_type=jnp.float32)
        sc_ref[comp] = acc


def _run_node_kernel(nf_t, attrs, w10, w11, wsk0_t, wsk1_t):
    grid = (N_NODES // NB_BLK,)
    return pl.pallas_call(
        _node_kernel,
        grid=grid,
        in_specs=[
            pl.BlockSpec((4, NB_BLK, C), lambda i: (0, i, 0)),
            pl.BlockSpec((NB_BLK, NSP), lambda i: (i, 0)),
            pl.BlockSpec((C, C), lambda i: (0, 0)),
            pl.BlockSpec((C, C), lambda i: (0, 0)),
            pl.BlockSpec((NSP, C, C), lambda i: (0, 0, 0)),
            pl.BlockSpec((NSP, C, C), lambda i: (0, 0, 0)),
        ],
        out_specs=[
            pl.BlockSpec((4, NB_BLK, C), lambda i: (0, i, 0)),
            pl.BlockSpec((G, NB_BLK, 4 * CG), lambda i: (0, i, 0)),
        ],
        out_shape=[
            jax.ShapeDtypeStruct((4, N_NODES, C), jnp.float32),
            jax.ShapeDtypeStruct((G, N_NODES, 4 * CG), jnp.float32),
        ],
    )(nf_t, attrs, w10, w11, wsk0_t, wsk1_t)


def _edge_kernel(ef_ref, a0_ref, a1_ref, a2_ref, a3_ref, tpw_ref):
    h = _silu(jnp.dot(ef_ref[...], a0_ref[...],
                      preferred_element_type=jnp.float32))
    h = _silu(jnp.dot(h, a1_ref[...], preferred_element_type=jnp.float32))
    h = _silu(jnp.dot(h, a2_ref[...], preferred_element_type=jnp.float32))
    t = jnp.dot(h, a3_ref[...], preferred_element_type=jnp.float32)
    for g in range(G):
        tpw_ref[g] = t[:, g * NPATHS * CG:(g + 1) * NPATHS * CG]


def _run_edge_kernel(ef, a0t, a1t, a2t, a3pt):
    grid = (N_EDGES // EB_BLK,)
    return pl.pallas_call(
        _edge_kernel,
        grid=grid,
        in_specs=[
            pl.BlockSpec((EB_BLK, NB), lambda i: (i, 0)),
            pl.BlockSpec((NB, HID), lambda i: (0, 0)),
            pl.BlockSpec((HID, HID), lambda i: (0, 0)),
            pl.BlockSpec((HID, HID), lambda i: (0, 0)),
            pl.BlockSpec((HID, NPATHS * C), lambda i: (0, 0)),
        ],
        out_specs=[
            pl.BlockSpec((G, EB_BLK, NPATHS * CG), lambda i: (0, i, 0)),
        ],
        out_shape=[
            jax.ShapeDtypeStruct((G, N_EDGES, NPATHS * CG), jnp.float32),
        ],
    )(ef, a0t, a1t, a2t, a3pt)[0]


B_EDGE = 40
B_IDX = 48
EPT = N_EDGES // 16
NCHUNK = EPT // B_EDGE
N_PAD = 10112
RPT = N_PAD // 16
ZR = 8


def _sc_body(ytab_h, tpw_h, rec_h, idxj_h, idxi_h, out_h,
             acc, rec_v, tpw_v, idxj_v, idxi_v, idx2_v, z_v, p_v, zero_v,
             sem_in0, sem_in1, sem_j0, sem_j1, sem_g):
    cid = lax.axis_index("c")
    sid = lax.axis_index("s")
    sem_in = (sem_in0, sem_in1)
    sem_j = (sem_j0, sem_j1)

    def zinit(t, _):
        r = t // SEC
        k = t % SEC
        zero_v[r, pl.ds(k * CG, CG)] = jnp.zeros((CG,), jnp.float32)
        return 0
    lax.fori_loop(0, ZR * SEC, zinit, 0)
    for s in range(2):
        idxj_v[s, pl.ds(2 * 16, 16)] = jnp.zeros((16,), jnp.int32)

    def issue_inputs(g, ch, s):
        base = sid * EPT + lax.rem(ch, NCHUNK) * B_EDGE
        pltpu.async_copy(rec_h.at[pl.ds(base, B_EDGE)], rec_v.at[s],
                         sem_in[s])
        pltpu.async_copy(tpw_h.at[g, pl.ds(base, B_EDGE)], tpw_v.at[s],
                         sem_in[s])
        pltpu.async_copy(idxi_h.at[pl.ds(base, B_EDGE)], idxi_v.at[s],
                         sem_in[s])
        pltpu.async_copy(idxj_h.at[pl.ds(base, B_EDGE)],
                         idxj_v.at[s, pl.ds(0, B_EDGE)], sem_j[s])

    def wait_inputs(s):
        for c in (
            pltpu.make_async_copy(rec_h.at[pl.ds(0, B_EDGE)], rec_v.at[s],
                                  sem_in[s]),
            pltpu.make_async_copy(tpw_h.at[0, pl.ds(0, B_EDGE)], tpw_v.at[s],
                                  sem_in[s]),
            pltpu.make_async_copy(idxi_h.at[pl.ds(0, B_EDGE)], idxi_v.at[s],
                                  sem_in[s]),
        ):
            c.wait()

    def wait_idxj(s):
        pltpu.make_async_copy(idxj_h.at[pl.ds(0, B_EDGE)],
                              idxj_v.at[s, pl.ds(0, B_EDGE)], sem_j[s]).wait()

    for gi in range(G // 2):
        g = cid * (G // 2) + gi

        for r5 in range(RPT // ZR):
            pltpu.sync_copy(zero_v, acc.at[pl.ds(sid * RPT + r5 * ZR, ZR)])
        plsc.subcore_barrier()

        issue_inputs(g, 0, 0)

        def chunk2(ch2, _):
            for par in range(2):
                ch = ch2 * 2 + par
                wait_idxj(par)

                def addb(k, _):
                    idx2_v[pl.ds(k * 16, 16)] = \
                        idxj_v[par, pl.ds(k * 16, 16)] + g * N_NODES
                    return 0
                lax.fori_loop(0, B_IDX // 16, addb, 0)
                cg_ = pltpu.async_copy(ytab_h.at[idx2_v], z_v, sem_g)
                issue_inputs(g, ch + 1, 1 - par)
                wait_inputs(par)
                cg_.wait()

                def edge(e, _):
                    recrow = rec_v[par, e, pl.ds(0, 16)]
                    a0 = recrow[0]
                    a10 = recrow[1]
                    a11 = recrow[2]
                    a12 = recrow[3]
                    z0 = z_v[e, pl.ds(0, CG)]
                    z10 = z_v[e, pl.ds(CG, CG)]
                    z11 = z_v[e, pl.ds(2 * CG, CG)]
                    z12 = z_v[e, pl.ds(3 * CG, CG)]
                    w0 = tpw_v[par, e, pl.ds(0, CG)]
                    w1 = tpw_v[par, e, pl.ds(CG, CG)]
                    w2 = tpw_v[par, e, pl.ds(2 * CG, CG)]
                    w3 = tpw_v[par, e, pl.ds(3 * CG, CG)]
                    w4 = tpw_v[par, e, pl.ds(4 * CG, CG)]
                    p_v[e, pl.ds(0, CG)] = w0 * z0 * a0
                    p_v[e, pl.ds(CG, CG)] = w3 * (z10 * a10 + z11 * a11
                                                  + z12 * a12)
                    u = w1 * z0
                    p_v[e, pl.ds(2 * CG, CG)] = u * a10
                    p_v[e, pl.ds(3 * CG, CG)] = u * a11
                    p_v[e, pl.ds(4 * CG, CG)] = u * a12
                    w2s = w2 * a0
                    p_v[e, pl.ds(5 * CG, CG)] = w2s * z10
                    p_v[e, pl.ds(6 * CG, CG)] = w2s * z11
                    p_v[e, pl.ds(7 * CG, CG)] = w2s * z12
                    t0 = (z10 * recrow[4] + z11 * recrow[7]
                          + z12 * recrow[10])
                    t1 = (z10 * recrow[5] + z11 * recrow[8]
                          + z12 * recrow[11])
                    t2 = (z10 * recrow[6] + z11 * recrow[9]
                          + z12 * recrow[12])
                    p_v[e, pl.ds(8 * CG, CG)] = w4 * t0
                    p_v[e, pl.ds(9 * CG, CG)] = w4 * t1
                    p_v[e, pl.ds(10 * CG, CG)] = w4 * t2
                    return 0
                lax.fori_loop(0, B_EDGE, edge, 0)
                pltpu.sync_copy(p_v, acc.at[idxi_v.at[par]], add=True)
            return 0
        lax.fori_loop(0, NCHUNK // 2, chunk2, 0)
        wait_idxj(0)
        wait_inputs(0)
        plsc.subcore_barrier()
        pltpu.sync_copy(acc.at[pl.ds(sid * RPT, RPT)],
                        out_h.at[g, pl.ds(sid * RPT, RPT)])
        if gi < G // 2 - 1:
            plsc.subcore_barrier()


def _run_sc_stage(ytab_flat, tpw, rec, idxj, idxi):
    mesh = plsc.VectorSubcoreMesh(core_axis_name="c", subcore_axis_name="s")
    fn = functools.partial(
        pl.kernel,
        out_type=jax.ShapeDtypeStruct((G, N_PAD, ROW), jnp.float32),
        mesh=mesh,
        compiler_params=pltpu.CompilerParams(use_tc_tiling_on_sc=False),
        scratch_types=[
            pltpu.VMEM_SHARED((N_PAD, ROW), jnp.float32),
            pltpu.VMEM((2, B_EDGE, 16), jnp.float32),
            pltpu.VMEM((2, B_EDGE, NPATHS * CG), jnp.float32),
            pltpu.VMEM((2, B_IDX), jnp.int32),
            pltpu.VMEM((2, B_EDGE), jnp.int32),
            pltpu.VMEM((B_IDX,), jnp.int32),
            pltpu.VMEM((B_IDX, 4 * CG), jnp.float32),
            pltpu.VMEM((B_EDGE, ROW), jnp.float32),
            pltpu.VMEM((ZR, ROW), jnp.float32),
            pltpu.SemaphoreType.DMA,
            pltpu.SemaphoreType.DMA,
            pltpu.SemaphoreType.DMA,
            pltpu.SemaphoreType.DMA,
            pltpu.SemaphoreType.DMA,
        ],
    )(_sc_body)
    return fn(ytab_flat, tpw, rec, idxj, idxi)


def _out_kernel(msg_ref, wbig_ref, o_ref):
    acc = jnp.zeros((NB_BLK, OUT * 4), jnp.float32)
    for g in range(G):
        acc += jnp.dot(msg_ref[g], wbig_ref[g],
                       preferred_element_type=jnp.float32)
    o_ref[...] = acc


def _run_out_kernel(msg, wbig):
    grid = (N_NODES // NB_BLK,)
    return pl.pallas_call(
        _out_kernel,
        grid=grid,
        in_specs=[
            pl.BlockSpec((G, NB_BLK, ROW), lambda i: (0, i, 0)),
            pl.BlockSpec((G, ROW, OUT * 4), lambda i: (0, 0, 0)),
        ],
        out_specs=[pl.BlockSpec((NB_BLK, OUT * 4), lambda i: (i, 0))],
        out_shape=[jax.ShapeDtypeStruct((N_NODES, OUT * 4), jnp.float32)],
    )(msg, wbig)[0]


def _build_wbig(W2_0, W2_1):
    s0 = 1.0 / (np.sqrt(2 * C) * AVG)
    s1 = 1.0 / (np.sqrt(3 * C) * AVG)
    wb = jnp.zeros((G, ROW, OUT * 4), jnp.float32)
    part = W2_0[:, :C].reshape(OUT, G, CG).transpose(1, 2, 0) * s0
    wb = wb.at[:, 0:CG, 0::4].set(part)
    part = W2_0[:, C:].reshape(OUT, G, CG).transpose(1, 2, 0) * s0
    wb = wb.at[:, CG:2 * CG, 0::4].set(part)
    for pi in range(3):
        part = (W2_1[:, pi * C:(pi + 1) * C]
                .reshape(OUT, G, CG).transpose(1, 2, 0) * s1)
        for d in range(3):
            sec = 2 + pi * 3 + d
            wb = wb.at[:, sec * CG:(sec + 1) * CG, (1 + d)::4].set(part)
    return wb


def _prep(node_feats, edge_attrs, idx_i, idx_j,
          W1_0, W1_1, A0, A1, A2, A3, W2_0, W2_1, Wsk0, Wsk1):
    nf_t = jnp.transpose(node_feats, (2, 0, 1))
    rec = jnp.pad(edge_attrs, ((0, 0), (0, 3)))
    idxj = idx_j.astype(jnp.int32)
    idxi = idx_i.astype(jnp.int32)
    w10 = W1_0.T / np.sqrt(C)
    w11 = W1_1.T / np.sqrt(C)
    norm_sk = 1.0 / np.sqrt(C * NSP)
    wsk0_t = jnp.transpose(Wsk0, (2, 1, 0)) * norm_sk
    wsk1_t = jnp.transpose(Wsk1, (2, 1, 0)) * norm_sk
    a0t = A0.T / np.sqrt(NB)
    a1t = A1.T / np.sqrt(HID)
    a2t = A2.T / np.sqrt(HID)
    a3p = (A3.reshape(NPATHS, G, CG, HID)
           .transpose(1, 0, 2, 3).reshape(NPATHS * C, HID))
    a3pt = a3p.T / np.sqrt(HID)
    wbig = _build_wbig(W2_0, W2_1)
    return (nf_t, rec, idxj, idxi, w10, w11, wsk0_t, wsk1_t,
            a0t, a1t, a2t, a3pt, wbig)


def kernel(node_attrs, node_feats, edge_attrs, edge_feats, idx_i, idx_j,
           W1_0, W1_1, A0, A1, A2, A3, W2_0, W2_1, Wsk0, Wsk1):
    (nf_t, rec, idxj, idxi, w10, w11, wsk0_t, wsk1_t,
     a0t, a1t, a2t, a3pt, wbig) = _prep(
        node_feats, edge_attrs, idx_i, idx_j,
        W1_0, W1_1, A0, A1, A2, A3, W2_0, W2_1, Wsk0, Wsk1)

    sc_t, ytab = _run_node_kernel(nf_t, node_attrs, w10, w11, wsk0_t, wsk1_t)
    tpw = _run_edge_kernel(edge_feats, a0t, a1t, a2t, a3pt)
    ytab_flat = ytab.reshape(G * N_NODES, 4 * CG)
    msg = _run_sc_stage(ytab_flat, tpw, rec, idxj, idxi)
    oflat = _run_out_kernel(msg, wbig)

    message = oflat.reshape(N_NODES, OUT, 4)
    sc = jnp.transpose(sc_t, (1, 2, 0))
    return (message, sc)
